# baseline (speedup 1.0000x reference)
import jax
import jax.numpy as jnp
from jax.experimental import pallas as pl
from jax.experimental.pallas import tpu as pltpu

_BLK_A = 400
_BLK_D = 200


def _xw_kernel(x_ref, w_ref, o_ref):
    o_ref[...] = jnp.dot(x_ref[...], w_ref[...],
                         preferred_element_type=jnp.float32)


def _make_fused(nb_a, lat):
    def _fused_kernel(a_ref, xw_ref, wc_ref, eps_ref,
                      muls_ref, ahat_ref,
                      hw_s, z_s, zt_s):
        i = pl.program_id(0)
        blk_a = a_ref.shape[0]
        blk_d = ahat_ref.shape[0]

        @pl.when(i < nb_a)
        def _():
            t = jnp.maximum(
                jnp.dot(a_ref[...], xw_ref[...],
                        preferred_element_type=jnp.float32), 0.0)
            hw_s[pl.ds(i * blk_a, blk_a), :] = jnp.dot(
                t, wc_ref[...], preferred_element_type=jnp.float32)

        @pl.when((i >= nb_a) & (i < 2 * nb_a))
        def _():
            out = jnp.dot(a_ref[...], hw_s[...],
                          preferred_element_type=jnp.float32)
            mu = out[:, :lat]
            ls = out[:, lat:]
            muls_ref[...] = out
            z_s[pl.ds((2 * nb_a - 1 - i) * blk_a, blk_a), :] = (
                mu + eps_ref[...] * jnp.exp(ls))

        @pl.when(i == 2 * nb_a)
        def _():
            zt_s[...] = z_s[...].T

        @pl.when(i >= 2 * nb_a)
        def _():
            zr = z_s[pl.ds((i - 2 * nb_a) * blk_d, blk_d), :]
            logits = jnp.dot(zr, zt_s[...],
                             preferred_element_type=jnp.float32)
            ahat_ref[...] = jax.nn.sigmoid(logits)

    return _fused_kernel


def kernel(X, A_tilde, epsilon, W1, W_mu, W_logsigma):
    n, f_in = X.shape
    hid = W1.shape[1]
    lat = W_mu.shape[1]
    nb_a = n // _BLK_A
    nb_d = n // _BLK_D
    w_cat = jnp.concatenate([W_mu, W_logsigma], axis=1)

    xw = pl.pallas_call(
        _xw_kernel,
        out_shape=jax.ShapeDtypeStruct((n, hid), jnp.float32),
    )(X, W1)

    def a_idx(i):
        return (jnp.where(i < nb_a, i,
                          jnp.maximum(2 * nb_a - 1 - i, 0)), 0)

    def l2_idx(i):
        return (jnp.clip(2 * nb_a - 1 - i, 0, nb_a - 1), 0)

    def dec_idx(i):
        return (jnp.clip(i - 2 * nb_a, 0, nb_d - 1), 0)

    muls, a_hat = pl.pallas_call(
        _make_fused(nb_a, lat),
        grid=(2 * nb_a + nb_d,),
        in_specs=[
            pl.BlockSpec((_BLK_A, n), a_idx),
            pl.BlockSpec((n, hid), lambda i: (0, 0)),
            pl.BlockSpec((hid, 2 * lat), lambda i: (0, 0)),
            pl.BlockSpec((_BLK_A, lat), l2_idx),
        ],
        out_specs=[
            pl.BlockSpec((_BLK_A, 2 * lat), l2_idx),
            pl.BlockSpec((_BLK_D, n), dec_idx),
        ],
        out_shape=[
            jax.ShapeDtypeStruct((n, 2 * lat), jnp.float32),
            jax.ShapeDtypeStruct((n, n), jnp.float32),
        ],
        scratch_shapes=[
            pltpu.VMEM((n, 2 * lat), jnp.float32),
            pltpu.VMEM((n, lat), jnp.float32),
            pltpu.VMEM((lat, n), jnp.float32),
        ],
        compiler_params=pltpu.CompilerParams(
            dimension_semantics=("arbitrary",),
            vmem_limit_bytes=67000000),
    )(A_tilde, xw, w_cat, epsilon)

    return (a_hat, muls[:, :lat], muls[:, lat:])


# speedup vs baseline: 1.0132x; 1.0132x over previous
import jax
import jax.numpy as jnp
from jax.experimental import pallas as pl
from jax.experimental.pallas import tpu as pltpu

_BLK_A = 400
_BLK_D = 200


def _xw_kernel(x_ref, w_ref, o_ref):
    o_ref[...] = jnp.dot(x_ref[...], w_ref[...],
                         preferred_element_type=jnp.float32)


def _make_fused(nb_a, lat):
    def _fused_kernel(a_ref, xw_ref, wc_ref, eps_ref,
                      mu_ref, ls_ref, ahat_ref,
                      hw_s, z_s, zt_s):
        i = pl.program_id(0)
        blk_a = a_ref.shape[0]
        blk_d = ahat_ref.shape[0]

        @pl.when(i < nb_a)
        def _():
            t = jnp.maximum(
                jnp.dot(a_ref[...], xw_ref[...],
                        preferred_element_type=jnp.float32), 0.0)
            hw_s[pl.ds(i * blk_a, blk_a), :] = jnp.dot(
                t, wc_ref[...], preferred_element_type=jnp.float32)

        @pl.when((i >= nb_a) & (i < 2 * nb_a))
        def _():
            out = jnp.dot(a_ref[...], hw_s[...],
                          preferred_element_type=jnp.float32)
            mu = out[:, :lat]
            ls = out[:, lat:]
            mu_ref[...] = mu
            ls_ref[...] = ls
            z_s[pl.ds((2 * nb_a - 1 - i) * blk_a, blk_a), :] = (
                mu + eps_ref[...] * jnp.exp(ls))

        @pl.when(i == 2 * nb_a)
        def _():
            zt_s[...] = z_s[...].T

        @pl.when(i >= 2 * nb_a)
        def _():
            zr = z_s[pl.ds((i - 2 * nb_a) * blk_d, blk_d), :]
            logits = jnp.dot(zr, zt_s[...],
                             preferred_element_type=jnp.float32)
            ahat_ref[...] = jax.nn.sigmoid(logits)

    return _fused_kernel


def kernel(X, A_tilde, epsilon, W1, W_mu, W_logsigma):
    n, f_in = X.shape
    hid = W1.shape[1]
    lat = W_mu.shape[1]
    nb_a = n // _BLK_A
    nb_d = n // _BLK_D
    w_cat = jnp.concatenate([W_mu, W_logsigma], axis=1)

    xw = pl.pallas_call(
        _xw_kernel,
        out_shape=jax.ShapeDtypeStruct((n, hid), jnp.float32),
    )(X, W1)

    def a_idx(i):
        return (jnp.where(i < nb_a, i,
                          jnp.maximum(2 * nb_a - 1 - i, 0)), 0)

    def l2_idx(i):
        return (jnp.clip(2 * nb_a - 1 - i, 0, nb_a - 1), 0)

    def dec_idx(i):
        return (jnp.clip(i - 2 * nb_a, 0, nb_d - 1), 0)

    mu, logsigma, a_hat = pl.pallas_call(
        _make_fused(nb_a, lat),
        grid=(2 * nb_a + nb_d,),
        in_specs=[
            pl.BlockSpec((_BLK_A, n), a_idx),
            pl.BlockSpec((n, hid), lambda i: (0, 0)),
            pl.BlockSpec((hid, 2 * lat), lambda i: (0, 0)),
            pl.BlockSpec((_BLK_A, lat), l2_idx),
        ],
        out_specs=[
            pl.BlockSpec((_BLK_A, lat), l2_idx),
            pl.BlockSpec((_BLK_A, lat), l2_idx),
            pl.BlockSpec((_BLK_D, n), dec_idx),
        ],
        out_shape=[
            jax.ShapeDtypeStruct((n, lat), jnp.float32),
            jax.ShapeDtypeStruct((n, lat), jnp.float32),
            jax.ShapeDtypeStruct((n, n), jnp.float32),
        ],
        scratch_shapes=[
            pltpu.VMEM((n, 2 * lat), jnp.float32),
            pltpu.VMEM((n, lat), jnp.float32),
            pltpu.VMEM((lat, n), jnp.float32),
        ],
        compiler_params=pltpu.CompilerParams(
            dimension_semantics=("arbitrary",),
            vmem_limit_bytes=67000000),
    )(A_tilde, xw, w_cat, epsilon)

    return (a_hat, mu, logsigma)
